# baseline (speedup 1.0000x reference)
import functools

import jax
import jax.numpy as jnp
from jax import lax
from jax.experimental import pallas as pl
from jax.experimental.pallas import tpu as pltpu
from jax.experimental.pallas import tpu_sc as plsc

B = 16384
D = 64
NC = 2
NS = 16
NW = NC * NS
BPW = B // NW
L = 16
NCOLS = 7813
SCRATCH_ROWS = B + 128
RING = 8


def _sweep_body(idx_hbm, t_hbm, scratch_hbm,
                alli, hits, hpos, blk0, blk1, stage, idxrows,
                bsem0, bsem1, ssem0, ssem1, ssem2, ssem3,
                ssem4, ssem5, ssem6, ssem7):
    ssems = (ssem0, ssem1, ssem2, ssem3, ssem4, ssem5, ssem6, ssem7)
    wid = lax.axis_index("s") * NC + lax.axis_index("c")
    cstart = wid * 244 + jnp.minimum(wid, 5)
    csize = 244 + (wid < 5).astype(jnp.int32)

    pltpu.sync_copy(idx_hbm, alli.at[pl.ds(0, B)])

    lanes = lax.iota(jnp.int32, L)

    def compact(t, cursor):
        uv = alli[pl.ds(t * L, L)]
        cv = uv >> 7
        m = (cv >= cstart) & (cv < cstart + csize)
        mi = m.astype(jnp.int32)
        incl = plsc.cumsum(mi)
        pos = cursor + incl - mi
        plsc.store_scatter(hits, [pos], uv, mask=m)
        plsc.store_scatter(hpos, [pos], t * L + lanes, mask=m)
        return cursor + incl[L - 1]

    count = lax.fori_loop(0, B // L, compact, jnp.int32(0))
    ngrp = (count + (L * RING - 1)) >> 7

    blks = (blk0, blk1)
    bsems = (bsem0, bsem1)

    def fire(cb, p):
        c = cstart + jnp.minimum(cb, csize - 1)
        off = pl.multiple_of(c * 128, 128)
        pltpu.async_copy(t_hbm.at[:, pl.ds(off, 128)], blks[p], bsems[p])

    fire(0, 0)
    fire(1, 1)

    def process_block(cb, flags, p):
        pltpu.make_async_copy(t_hbm.at[:, pl.ds(0, 128)],
                              blks[p], bsems[p]).wait()
        c = cstart + jnp.minimum(cb, csize - 1)

        def scan_group(g, flags):
            flags = list(flags)
            for s in range(RING):
                h = g * RING + s
                hv = hits[pl.ds(h * L, L)]
                valid = (h * L + lanes) < count
                hm = ((hv >> 7) == c) & valid
                matched = jnp.sum(hm.astype(jnp.int32))

                @pl.when(matched > 0)
                def _(s=s, hv=hv, hm=hm, h=h, flag=flags[s]):
                    @pl.when(flag > 0)
                    def _():
                        pltpu.make_async_copy(
                            scratch_hbm.at[pl.ds(0, L), :],
                            stage.at[s], ssems[s]).wait()
                    lvec = hv & 127
                    slotv = jnp.full((L,), s, jnp.int32)
                    for d in range(D):
                        val = plsc.load_gather(
                            blks[p], [jnp.full((L,), d, jnp.int32), lvec])
                        plsc.store_scatter(
                            stage,
                            [slotv, lanes, jnp.full((L,), d, jnp.int32)],
                            val)
                    ridx = jnp.where(hm, hpos[pl.ds(h * L, L)],
                                     B + wid * 4 + (lanes & 3))
                    plsc.store_scatter(idxrows, [slotv, lanes], ridx)
                    pltpu.async_copy(stage.at[s],
                                     scratch_hbm.at[idxrows.at[s]], ssems[s])

                flags[s] = jnp.where(matched > 0, jnp.int32(1), flags[s])
            return tuple(flags)

        flags = lax.fori_loop(0, ngrp, scan_group, flags)
        fire(cb + 2, p)
        return flags

    def pair_body(r, flags):
        flags = process_block(2 * r, flags, 0)
        flags = process_block(2 * r + 1, flags, 1)
        return flags

    npairs = (csize + 1) >> 1
    flags = lax.fori_loop(0, npairs, pair_body,
                          tuple(jnp.int32(0) for _ in range(RING)))

    for p in (0, 1):
        pltpu.make_async_copy(t_hbm.at[:, pl.ds(0, 128)],
                              blks[p], bsems[p]).wait()

    for s in range(RING):
        @pl.when(flags[s] > 0)
        def _(s=s):
            pltpu.make_async_copy(scratch_hbm.at[pl.ds(0, L), :],
                                  stage.at[s], ssems[s]).wait()


def _dot_body(srow_u_hbm, srow_v_hbm, out_hbm, urows, vrows, outv, sem):
    wid = lax.axis_index("s") * NC + lax.axis_index("c")
    base = wid * BPW
    lanes = lax.iota(jnp.int32, L)
    HALF = BPW // 2

    for half in range(2):
        hb = half * HALF
        pltpu.sync_copy(srow_u_hbm.at[pl.ds(base + hb, HALF), :], urows)
        pltpu.sync_copy(srow_v_hbm.at[pl.ds(base + hb, HALF), :], vrows)

        def chunk(cidx, carry):
            q = jnp.zeros((L,), jnp.float32)
            for r in range(L):
                e = cidx * L + r
                acc = None
                for j in range(D // L):
                    u = urows[e, pl.ds(j * L, L)]
                    v = vrows[e, pl.ds(j * L, L)]
                    prod = u * v
                    acc = prod if acc is None else acc + prod
                q = jnp.where(lanes == r, jnp.sum(acc), q)
            outv[pl.ds(hb + cidx * L, L)] = 1.0 / (1.0 + jnp.exp(-q))
            return carry

        lax.fori_loop(0, HALF // L, chunk, 0)

    pltpu.sync_copy(outv, out_hbm.at[pl.ds(base, BPW)])


def kernel(user, item, user_table, item_table):
    mesh = plsc.VectorSubcoreMesh(core_axis_name="c", subcore_axis_name="s")
    cp = pltpu.CompilerParams(needs_layout_passes=False)

    sweep = functools.partial(
        pl.kernel,
        out_type=jax.ShapeDtypeStruct((SCRATCH_ROWS, 128), jnp.float32),
        mesh=mesh,
        compiler_params=cp,
        scratch_types=[
            pltpu.VMEM((B,), jnp.int32),
            pltpu.VMEM((B,), jnp.int32),
            pltpu.VMEM((B,), jnp.int32),
            pltpu.VMEM((D, 128), jnp.float32),
            pltpu.VMEM((D, 128), jnp.float32),
            pltpu.VMEM((RING, L, 128), jnp.float32),
            pltpu.VMEM((RING, L), jnp.int32),
        ] + [pltpu.SemaphoreType.DMA] * (2 + RING),
    )(_sweep_body)

    dot = functools.partial(
        pl.kernel,
        out_type=jax.ShapeDtypeStruct((B,), jnp.float32),
        mesh=mesh,
        compiler_params=cp,
        scratch_types=[
            pltpu.VMEM((BPW // 2, 128), jnp.float32),
            pltpu.VMEM((BPW // 2, 128), jnp.float32),
            pltpu.VMEM((BPW,), jnp.float32),
            pltpu.SemaphoreType.DMA,
        ],
    )(_dot_body)

    su = sweep(user, user_table.T)
    sv = sweep(item, item_table.T)
    return dot(su, sv)


# speedup vs baseline: 6.4302x; 6.4302x over previous
import functools

import jax
import jax.numpy as jnp
from jax import lax
from jax.experimental import pallas as pl
from jax.experimental.pallas import tpu as pltpu
from jax.experimental.pallas import tpu_sc as plsc

B = 16384
D = 64
NC = 2
NS = 16
NW = NC * NS
BPW = B // NW
L = 16
WAVES = BPW // 2


DEPTH = 6
MAIN = (BPW // DEPTH) * DEPTH


def _mf_body(user_hbm, item_hbm, ut_hbm, it_hbm, out_hbm,
             uidx, iidx,
             ub0, ub1, ub2, ub3, ub4, ub5, vb0, vb1, vb2, vb3, vb4, vb5,
             outv, sem0, sem1, sem2, sem3, sem4, sem5):
    wid = lax.axis_index("s") * NC + lax.axis_index("c")
    base = wid * BPW

    ub = (ub0, ub1, ub2, ub3, ub4, ub5)
    vb = (vb0, vb1, vb2, vb3, vb4, vb5)
    sems = (sem0, sem1, sem2, sem3, sem4, sem5)

    pltpu.sync_copy(user_hbm.at[pl.ds(base, BPW)], uidx.at[pl.ds(0, BPW)])
    pltpu.sync_copy(item_hbm.at[pl.ds(base, BPW)], iidx.at[pl.ds(0, BPW)])

    lanes = lax.iota(jnp.int32, L)
    zeros = jnp.zeros((L,), jnp.float32)

    def fire(w, parity):
        i0 = jnp.minimum(w, BPW - 1)
        uv = uidx[pl.ds(i0, L)]
        iv = iidx[pl.ds(i0, L)]
        cu = pl.multiple_of((uv[0] >> 7) * 128, 128)
        cv = pl.multiple_of((iv[0] >> 7) * 128, 128)
        pltpu.async_copy(ut_hbm.at[:, pl.ds(cu, 128)], ub[parity],
                         sems[parity])
        pltpu.async_copy(it_hbm.at[:, pl.ds(cv, 128)], vb[parity],
                         sems[parity])

    for s in range(DEPTH):
        fire(s, s)

    def do_wave(w, q, s, refire):
        for _ in range(2):
            pltpu.make_async_copy(ut_hbm.at[:, pl.ds(0, 128)],
                                  ub[0], sems[s]).wait()
        uv = uidx[pl.ds(w, L)]
        iv = iidx[pl.ds(w, L)]
        lu = jnp.full((L,), uv[0] & 127, jnp.int32)
        lv = jnp.full((L,), iv[0] & 127, jnp.int32)
        acc = None
        for j in range(D // L):
            rows = lanes + (j * L)
            uc = plsc.load_gather(ub[s], [rows, lu])
            vc = plsc.load_gather(vb[s], [rows, lv])
            prod = uc * vc
            acc = prod if acc is None else acc + prod
        q = jnp.where(lanes == (w & 15), jnp.sum(acc), q)
        if refire:
            fire(w + DEPTH, s)
        flush = (w & 15) == 15
        @pl.when(flush)
        def _():
            outv[pl.ds((w >> 4) * L, L)] = 1.0 / (1.0 + jnp.exp(-q))
        return jnp.where(flush, zeros, q)

    def group_body(t, q):
        for s in range(DEPTH):
            q = do_wave(DEPTH * t + s, q, s, True)
        return q

    q = lax.fori_loop(0, MAIN // DEPTH, group_body, zeros)
    for w in range(MAIN, BPW):
        q = do_wave(jnp.int32(w), q, w % DEPTH, False)

    for w in range(BPW, MAIN + DEPTH):
        for _ in range(2):
            pltpu.make_async_copy(ut_hbm.at[:, pl.ds(0, 128)],
                                  ub[0], sems[w % DEPTH]).wait()

    pltpu.sync_copy(outv, out_hbm.at[pl.ds(base, BPW)])


def kernel(user, item, user_table, item_table):
    mesh = plsc.VectorSubcoreMesh(core_axis_name="c", subcore_axis_name="s")
    blk = lambda: pltpu.VMEM((D, 128), jnp.float32)
    run = functools.partial(
        pl.kernel,
        out_type=jax.ShapeDtypeStruct((B,), jnp.float32),
        mesh=mesh,
        compiler_params=pltpu.CompilerParams(needs_layout_passes=False),
        scratch_types=[
            pltpu.VMEM((BPW + L,), jnp.int32),
            pltpu.VMEM((BPW + L,), jnp.int32),
        ] + [blk() for _ in range(2 * DEPTH)] + [
            pltpu.VMEM((BPW,), jnp.float32),
        ] + [pltpu.SemaphoreType.DMA] * DEPTH,
    )(_mf_body)
    return run(user, item, user_table.T, item_table.T)
